# baseline (speedup 1.0000x reference)
import functools

import jax
import jax.numpy as jnp
from jax import lax
from jax.experimental import pallas as pl
from jax.experimental.pallas import tpu as pltpu
from jax.experimental.pallas import tpu_sc as plsc

B, C, H, W = 8, 64, 32, 32
TOK_PER_B = H * W
N_CODES = 8192
CODE_TILE = 512
N_CT = N_CODES // CODE_TILE

SC_NC, SC_NS = 2, 16
SC_NW = SC_NC * SC_NS
N_TOK = B * TOK_PER_B
TOK_PER_W = N_TOK // SC_NW


def _argmax_body(x_ref, cb_ref, out_ref, xn_ref, rmax_ref, ridx_ref):
    ct = pl.program_id(1)

    @pl.when(ct == 0)
    def _init():
        xb = x_ref[0]
        norm = jnp.sqrt(jnp.sum(xb * xb, axis=0, keepdims=True))
        xn_ref[...] = xb / jnp.maximum(norm, 1e-12)
        rmax_ref[...] = jnp.full((1, TOK_PER_B), -jnp.inf, jnp.float32)
        ridx_ref[...] = jnp.zeros((1, TOK_PER_B), jnp.int32)

    s = lax.dot_general(
        cb_ref[...], xn_ref[...],
        (((1,), (0,)), ((), ())),
        preferred_element_type=jnp.float32,
        precision=lax.Precision.DEFAULT,
    )
    tmax = jnp.max(s, axis=0, keepdims=True)
    row = lax.broadcasted_iota(jnp.int32, (CODE_TILE, TOK_PER_B), 0)
    targ = jnp.min(jnp.where(s == tmax, row, jnp.int32(2**30)),
                   axis=0, keepdims=True) + ct * CODE_TILE
    better = tmax > rmax_ref[...]
    ridx_ref[...] = jnp.where(better, targ, ridx_ref[...])
    rmax_ref[...] = jnp.where(better, tmax, rmax_ref[...])

    @pl.when(ct == N_CT - 1)
    def _emit():
        out_ref[...] = ridx_ref[...].reshape(1, 1, TOK_PER_B)


def _nearest_code(xr, codebook):
    return pl.pallas_call(
        _argmax_body,
        grid=(B, N_CT),
        in_specs=[
            pl.BlockSpec((1, C, TOK_PER_B), lambda b, ct: (b, 0, 0)),
            pl.BlockSpec((CODE_TILE, C), lambda b, ct: (ct, 0)),
        ],
        out_specs=pl.BlockSpec((1, 1, TOK_PER_B), lambda b, ct: (b, 0, 0)),
        out_shape=jax.ShapeDtypeStruct((B, 1, TOK_PER_B), jnp.int32),
        scratch_shapes=[
            pltpu.VMEM((C, TOK_PER_B), jnp.float32),
            pltpu.VMEM((1, TOK_PER_B), jnp.float32),
            pltpu.VMEM((1, TOK_PER_B), jnp.int32),
        ],
    )(xr, codebook)


@functools.cache
def _make_sc_gather():
    @functools.partial(
        pl.kernel,
        out_type=jax.ShapeDtypeStruct((N_TOK, C), jnp.float32),
        mesh=plsc.VectorSubcoreMesh(core_axis_name="c", subcore_axis_name="s"),
        compiler_params=pltpu.CompilerParams(use_tc_tiling_on_sc=False),
        scratch_types=[
            pltpu.VMEM((TOK_PER_W,), jnp.int32),
            pltpu.VMEM((TOK_PER_W, C), jnp.float32),
            pltpu.SemaphoreType.DMA,
        ],
    )
    def _sc_gather(table_hbm, idx_hbm, out_hbm, idx_v, rows_v, sem):
        wid = lax.axis_index("s") * SC_NC + lax.axis_index("c")
        base = wid * TOK_PER_W
        pltpu.sync_copy(idx_hbm.at[pl.ds(base, TOK_PER_W)], idx_v)
        pltpu.async_copy(table_hbm.at[idx_v], rows_v, sem).wait()
        pltpu.sync_copy(rows_v, out_hbm.at[pl.ds(base, TOK_PER_W)])

    return _sc_gather


def kernel(x, codebook):
    xr = x.reshape(B, C, TOK_PER_B)
    idx = _nearest_code(xr, codebook)
    rows = _make_sc_gather()(codebook, idx.reshape(N_TOK))
    q = rows.reshape(B, H, W, C)
    return jnp.transpose(q, (0, 3, 1, 2))


# speedup vs baseline: 1.0855x; 1.0855x over previous
import functools

import jax
import jax.numpy as jnp
from jax import lax
from jax.experimental import pallas as pl
from jax.experimental.pallas import tpu as pltpu
from jax.experimental.pallas import tpu_sc as plsc

B, C, H, W = 8, 64, 32, 32
TOK_PER_B = H * W
N_CODES = 8192
CODE_TILE = 512
N_CT = N_CODES // CODE_TILE
VR_PER_TILE = CODE_TILE // 8

SC_NC, SC_NS = 2, 16
SC_NW = SC_NC * SC_NS
N_TOK = B * TOK_PER_B
TOK_PER_W = N_TOK // SC_NW


def _argmax_body(x_ref, cb_ref, out_ref, xn_ref, rmax_ref, ridx_ref):
    ct = pl.program_id(1)

    @pl.when(ct == 0)
    def _init():
        xb = x_ref[0]
        norm = jnp.sqrt(jnp.sum(xb * xb, axis=0, keepdims=True))
        xn_ref[...] = xb / jnp.maximum(norm, 1e-12)

    s = lax.dot_general(
        cb_ref[...], xn_ref[...],
        (((1,), (0,)), ((), ())),
        preferred_element_type=jnp.float32,
        precision=lax.Precision.DEFAULT,
    )

    a, b = s[: CODE_TILE // 2], s[CODE_TILE // 2:]
    cmp = a >= b
    val = jnp.where(cmp, a, b)
    idx = jnp.where(cmp, jnp.int32(0), jnp.int32(1))
    rows = CODE_TILE // 2
    k = 1
    while rows > 8:
        rows //= 2
        a, b = val[:rows], val[rows:]
        ia, ib = idx[:rows], idx[rows:]
        cmp = a >= b
        val = jnp.where(cmp, a, b)
        idx = jnp.where(cmp, ia, ib + jnp.int32(1 << k))
        k += 1
    row = idx * 8 + lax.broadcasted_iota(jnp.int32, (8, TOK_PER_B), 0)
    sub = 8
    while sub > 1:
        sub //= 2
        a, b = val[:sub], val[sub:]
        ra, rb = row[:sub], row[sub:]
        win_a = (a > b) | ((a == b) & (ra < rb))
        val = jnp.where(win_a, a, b)
        row = jnp.where(win_a, ra, rb)
    code = row + ct * CODE_TILE

    prev_max = jnp.where(ct == 0, jnp.float32(-jnp.inf), rmax_ref[...])
    prev_idx = ridx_ref[...]
    better = val > prev_max
    ridx_ref[...] = jnp.where(better, code, prev_idx)
    rmax_ref[...] = jnp.where(better, val, prev_max)

    @pl.when(ct == N_CT - 1)
    def _emit():
        out_ref[...] = ridx_ref[...].reshape(1, 1, TOK_PER_B)


def _nearest_code(xr, cb_brev):
    return pl.pallas_call(
        _argmax_body,
        grid=(B, N_CT),
        in_specs=[
            pl.BlockSpec((1, C, TOK_PER_B), lambda b, ct: (b, 0, 0)),
            pl.BlockSpec((CODE_TILE, C), lambda b, ct: (ct, 0)),
        ],
        out_specs=pl.BlockSpec((1, 1, TOK_PER_B), lambda b, ct: (b, 0, 0)),
        out_shape=jax.ShapeDtypeStruct((B, 1, TOK_PER_B), jnp.int32),
        scratch_shapes=[
            pltpu.VMEM((C, TOK_PER_B), jnp.float32),
            pltpu.VMEM((1, TOK_PER_B), jnp.float32),
            pltpu.VMEM((1, TOK_PER_B), jnp.int32),
        ],
    )(xr, cb_brev)


def _bit_reverse_rows(cb):
    r = cb.reshape(N_CT, 2, 2, 2, 2, 2, 2, 8, C)
    r = r.transpose(0, 6, 5, 4, 3, 2, 1, 7, 8)
    return r.reshape(N_CODES, C)


@functools.cache
def _make_sc_gather():
    @functools.partial(
        pl.kernel,
        out_type=jax.ShapeDtypeStruct((N_TOK, C), jnp.float32),
        mesh=plsc.VectorSubcoreMesh(core_axis_name="c", subcore_axis_name="s"),
        compiler_params=pltpu.CompilerParams(use_tc_tiling_on_sc=False),
        scratch_types=[
            pltpu.VMEM((TOK_PER_W,), jnp.int32),
            pltpu.VMEM((TOK_PER_W, C), jnp.float32),
            pltpu.SemaphoreType.DMA,
        ],
    )
    def _sc_gather(table_hbm, idx_hbm, out_hbm, idx_v, rows_v, sem):
        wid = lax.axis_index("s") * SC_NC + lax.axis_index("c")
        base = wid * TOK_PER_W
        pltpu.sync_copy(idx_hbm.at[pl.ds(base, TOK_PER_W)], idx_v)
        pltpu.async_copy(table_hbm.at[idx_v], rows_v, sem).wait()
        pltpu.sync_copy(rows_v, out_hbm.at[pl.ds(base, TOK_PER_W)])

    return _sc_gather


def kernel(x, codebook):
    xr = x.reshape(B, C, TOK_PER_B)
    idx = _nearest_code(xr, _bit_reverse_rows(codebook))
    rows = _make_sc_gather()(codebook, idx.reshape(N_TOK))
    q = rows.reshape(B, H, W, C)
    return jnp.transpose(q, (0, 3, 1, 2))


# speedup vs baseline: 1.5480x; 1.4261x over previous
import functools

import jax
import jax.numpy as jnp
from jax import lax
from jax.experimental import pallas as pl
from jax.experimental.pallas import tpu as pltpu
from jax.experimental.pallas import tpu_sc as plsc

B, C, H, W = 8, 64, 32, 32
TOK_PER_B = H * W
N_CODES = 8192
CODE_TILE = 1024
N_CT = N_CODES // CODE_TILE
VR_PER_TILE = CODE_TILE // 8

SC_NC, SC_NS = 2, 16
SC_NW = SC_NC * SC_NS
N_TOK = B * TOK_PER_B
TOK_PER_W = N_TOK // SC_NW


def _tile_argmax(s):
    t = s.shape[-1]
    nvr = CODE_TILE // 8
    v3 = s.reshape(nvr // 2, 16, t)
    a, b = v3[:, :8, :], v3[:, 8:, :]
    cmp = a >= b
    val = jnp.where(cmp, a, b)
    idx = jnp.where(cmp, jnp.int32(0), jnp.int32(1))
    groups = nvr // 2
    k = 1
    while groups > 1:
        groups //= 2
        val = val.reshape(groups, 16, t)
        idx = idx.reshape(groups, 16, t)
        a, b = val[:, :8, :], val[:, 8:, :]
        ia, ib = idx[:, :8, :], idx[:, 8:, :]
        cmp = a >= b
        val = jnp.where(cmp, a, b)
        idx = jnp.where(cmp, ia, ib + jnp.int32(1 << k))
        k += 1
    val = val.reshape(8, t)
    idx = idx.reshape(8, t)
    row = idx * 8 + lax.broadcasted_iota(jnp.int32, (8, t), 0)
    sub = 8
    while sub > 1:
        sub //= 2
        a, b = val[:sub], val[sub:]
        ra, rb = row[:sub], row[sub:]
        win_a = (a > b) | ((a == b) & (ra < rb))
        val = jnp.where(win_a, a, b)
        row = jnp.where(win_a, ra, rb)
    return val, row


def _argmax_body(x_ref, cb_ref, out_ref, xn_ref, rmax_ref, ridx_ref):
    ct = pl.program_id(1)

    @pl.when(ct == 0)
    def _init():
        xb = x_ref[0]
        norm = jnp.sqrt(jnp.sum(xb * xb, axis=0, keepdims=True))
        xn_ref[...] = xb / jnp.maximum(norm, 1e-12)

    s = lax.dot_general(
        cb_ref[...], xn_ref[...],
        (((1,), (0,)), ((), ())),
        preferred_element_type=jnp.float32,
        precision=lax.Precision.DEFAULT,
    )

    val, row = _tile_argmax(s)
    code = row + ct * CODE_TILE

    prev_max = jnp.where(ct == 0, jnp.float32(-jnp.inf), rmax_ref[...])
    prev_idx = ridx_ref[...]
    better = val > prev_max
    ridx_ref[...] = jnp.where(better, code, prev_idx)
    rmax_ref[...] = jnp.where(better, val, prev_max)

    @pl.when(ct == N_CT - 1)
    def _emit():
        out_ref[...] = ridx_ref[...].reshape(1, 1, TOK_PER_B)


def _nearest_code(xr, codebook):
    return pl.pallas_call(
        _argmax_body,
        grid=(B, N_CT),
        in_specs=[
            pl.BlockSpec((1, C, TOK_PER_B), lambda b, ct: (b, 0, 0)),
            pl.BlockSpec((CODE_TILE, C), lambda b, ct: (ct, 0)),
        ],
        out_specs=pl.BlockSpec((1, 1, TOK_PER_B), lambda b, ct: (b, 0, 0)),
        out_shape=jax.ShapeDtypeStruct((B, 1, TOK_PER_B), jnp.int32),
        scratch_shapes=[
            pltpu.VMEM((C, TOK_PER_B), jnp.float32),
            pltpu.VMEM((1, TOK_PER_B), jnp.float32),
            pltpu.VMEM((1, TOK_PER_B), jnp.int32),
        ],
    )(xr, codebook)


@functools.cache
def _make_sc_gather():
    @functools.partial(
        pl.kernel,
        out_type=jax.ShapeDtypeStruct((N_TOK, C), jnp.float32),
        mesh=plsc.VectorSubcoreMesh(core_axis_name="c", subcore_axis_name="s"),
        compiler_params=pltpu.CompilerParams(use_tc_tiling_on_sc=False),
        scratch_types=[
            pltpu.VMEM((TOK_PER_W,), jnp.int32),
            pltpu.VMEM((TOK_PER_W, C), jnp.float32),
            pltpu.SemaphoreType.DMA,
        ],
    )
    def _sc_gather(table_hbm, idx_hbm, out_hbm, idx_v, rows_v, sem):
        wid = lax.axis_index("s") * SC_NC + lax.axis_index("c")
        base = wid * TOK_PER_W
        pltpu.sync_copy(idx_hbm.at[pl.ds(base, TOK_PER_W)], idx_v)
        pltpu.async_copy(table_hbm.at[idx_v], rows_v, sem).wait()
        pltpu.sync_copy(rows_v, out_hbm.at[pl.ds(base, TOK_PER_W)])

    return _sc_gather


def kernel(x, codebook):
    xr = x.reshape(B, C, TOK_PER_B)
    idx = _nearest_code(xr, codebook)
    rows = _make_sc_gather()(codebook, idx.reshape(N_TOK))
    q = rows.reshape(B, H, W, C)
    return jnp.transpose(q, (0, 3, 1, 2))
